# baseline (speedup 1.0000x reference)
import functools

import jax
import jax.numpy as jnp
from jax import lax
from jax.experimental import pallas as pl

N = 10000
E = 160000
D = 128
H = 8
DP = 16
TEMP = 4.0
BLK = 4000
GRID = E // BLK


def _group_mat(width, stride_same):
    ci = lax.broadcasted_iota(jnp.int32, (width, width), 0)
    cj = lax.broadcasted_iota(jnp.int32, (width, width), 1)
    if stride_same == "mod8":
        gi, gj = ci - (ci // 8) * 8, cj - (cj // 8) * 8
    else:
        gi, gj = ci // 8, cj // 8
    return (gi == gj).astype(jnp.float32)


def _edge_preacts(xi, xj, ef, rev, pi, pj,
                  WqT, bq, WkT, bk, M1q, M1k, b1f,
                  WaT, WbT, WcT, WdT, eu_b1,
                  W1pT, w1dr, dm_b1):
    dot = functools.partial(jnp.dot, preferred_element_type=jnp.float32)
    qlin = dot(xi, WqT) + bq
    klin = dot(ef, WkT) + bk
    a_pre = dot(qlin, M1q) + dot(klin, M1k) + b1f
    he_pre = (dot(xi, WaT) + dot(ef, WbT) + dot(rev, WcT) + dot(xj, WdT)
              + eu_b1)
    diff = pi - pj
    dist = jnp.sqrt(jnp.sum(diff * diff, axis=-1, keepdims=True) + 1e-12)
    hdm_pre = dot(diff, W1pT) + dist * w1dr + dm_b1
    return qlin, klin, a_pre, he_pre, hdm_pre


def _pass_a(xi_ref, xj_ref, ef_ref, rev_ref, pi_ref, pj_ref,
            WqT, bq, WkT, bk, M1q, M1k, b1f,
            WaT, WbT, WcT, WdT, eu_b1,
            W1pT, w1dr, dm_b1,
            s_hdm_ref, s_a_ref, s_he_ref):
    i = pl.program_id(0)

    @pl.when(i == 0)
    def _():
        s_hdm_ref[...] = jnp.zeros_like(s_hdm_ref)
        s_a_ref[...] = jnp.zeros_like(s_a_ref)
        s_he_ref[...] = jnp.zeros_like(s_he_ref)

    _, _, a_pre, he_pre, hdm_pre = _edge_preacts(
        xi_ref[...], xj_ref[...], ef_ref[...], rev_ref[...],
        pi_ref[...], pj_ref[...],
        WqT[...], bq[...], WkT[...], bk[...], M1q[...], M1k[...], b1f[...],
        WaT[...], WbT[...], WcT[...], WdT[...], eu_b1[...],
        W1pT[...], w1dr[...], dm_b1[...])

    s_hdm_ref[0:1, :] += jnp.sum(hdm_pre, axis=0, keepdims=True)
    s_hdm_ref[1:2, :] += jnp.sum(hdm_pre * hdm_pre, axis=0, keepdims=True)
    s_a_ref[0:1, :] += jnp.sum(a_pre, axis=0, keepdims=True)
    s_a_ref[1:2, :] += jnp.sum(a_pre * a_pre, axis=0, keepdims=True)
    s_he_ref[0:1, :] += jnp.sum(he_pre, axis=0, keepdims=True)
    s_he_ref[1:2, :] += jnp.sum(he_pre * he_pre, axis=0, keepdims=True)


def _pass_b(xi_ref, xj_ref, ef_ref, rev_ref, pi_ref, pj_ref,
            WqT, bq, WkT, bk, M1q, M1k, b1f,
            WaT, WbT, WcT, WdT, eu_b1,
            W1pT, w1dr, dm_b1,
            s_hdm, s_a, s_he,
            M2, b2f, WvT, bv, dm_W2T, dm_b2, eu_W2T, eu_b2,
            msg_ref, ue_ref, prob_ref):
    dot = functools.partial(jnp.dot, preferred_element_type=jnp.float32)
    qlin, klin, a_pre, he_pre, hdm_pre = _edge_preacts(
        xi_ref[...], xj_ref[...], ef_ref[...], rev_ref[...],
        pi_ref[...], pj_ref[...],
        WqT[...], bq[...], WkT[...], bk[...], M1q[...], M1k[...], b1f[...],
        WaT[...], WbT[...], WcT[...], WdT[...], eu_b1[...],
        W1pT[...], w1dr[...], dm_b1[...])

    inv_e = 1.0 / E
    hdm_m = s_hdm[0:1, :] * inv_e
    hdm_v = s_hdm[1:2, :] * inv_e - hdm_m * hdm_m
    hdm = jax.nn.relu((hdm_pre - hdm_m) * lax.rsqrt(hdm_v + 1e-5))
    dmask = jax.nn.sigmoid(dot(hdm, dm_W2T[...]) + dm_b2[...])

    g8 = _group_mat(256, "div8")
    a_cnt = 1.0 / (E * H)
    a_m = dot(s_a[0:1, :], g8) * a_cnt
    a_v = dot(s_a[1:2, :], g8) * a_cnt - a_m * a_m
    a_bn = jax.nn.relu((a_pre - a_m) * lax.rsqrt(a_v + 1e-5))
    att = dot(a_bn, M2[...]) + b2f[...]

    gh = _group_mat(128, "mod8")
    gmean = dot(att, gh) * (1.0 / DP)
    u = jnp.exp((att - gmean) * (1.0 / TEMP))
    denom = dot(u, gh)
    prob = u / denom
    prob_ref[...] = prob

    vlin = dot(xj_ref[...], WvT[...]) + bv[...]
    msg_ref[...] = prob * vlin * dmask

    he_m = s_he[0:1, :] * inv_e
    he_v = s_he[1:2, :] * inv_e - he_m * he_m
    he = jax.nn.relu((he_pre - he_m) * lax.rsqrt(he_v + 1e-5))
    ue_ref[...] = dot(he, eu_W2T[...]) + eu_b2[...]


def _node_kernel(x_ref, agg_ref, osum_ref, ocnt_ref, isum_ref, icnt_ref,
                 n1aT, n1bT, nu_b1, nu_W2T, nu_b2,
                 eaTa, eaTb, ea_b,
                 out_ref):
    dot = functools.partial(jnp.dot, preferred_element_type=jnp.float32)
    hn_pre = dot(x_ref[...], n1aT[...]) + dot(agg_ref[...], n1bT[...]) \
        + nu_b1[...]
    m = jnp.mean(hn_pre, axis=0, keepdims=True)
    v = jnp.mean(hn_pre * hn_pre, axis=0, keepdims=True) - m * m
    hn = jax.nn.relu((hn_pre - m) * lax.rsqrt(v + 1e-5))
    un = dot(hn, nu_W2T[...]) + nu_b2[...]
    om = osum_ref[...] / jnp.maximum(ocnt_ref[...], 1.0)
    im = isum_ref[...] / jnp.maximum(icnt_ref[...], 1.0)
    gate = jax.nn.sigmoid(dot(om, eaTa[...]) + dot(im, eaTb[...]) + ea_b[...])
    out_ref[...] = jax.nn.relu(un) * gate


def _full(shape):
    return pl.BlockSpec(shape, lambda i: (0,) * len(shape))


def kernel(x, edge_feature, node_positions, Wq, bq, Wk, bk, Wv, bv,
           dm_W1, dm_b1, dm_W2, dm_b2, att_W1, att_b1, att_W2, att_b2,
           eu_W1, eu_b1, eu_W2, eu_b2, ea_W, ea_b,
           nu_W1, nu_b1, nu_W2, nu_b2, edge_index):
    f32 = jnp.float32
    row = edge_index[0]
    col = edge_index[1]
    keys = row.astype(jnp.int64) * N + col.astype(jnp.int64)
    order = jnp.argsort(keys)
    skeys = keys[order]
    rkeys = col.astype(jnp.int64) * N + row.astype(jnp.int64)
    pos = jnp.clip(jnp.searchsorted(skeys, rkeys), 0, E - 1)
    found = skeys[pos] == rkeys
    rev_ef = jnp.where(found[:, None], edge_feature[jnp.clip(order[pos], 0, E - 1)], 0.0)
    x_i = x[row]
    x_j = x[col]
    p_i = node_positions[row]
    p_j = node_positions[col]

    eye8 = jnp.eye(8, dtype=f32)
    M1q = jnp.kron(att_W1[:, :DP].T, eye8)
    M1k = jnp.kron(att_W1[:, DP:].T, eye8)
    M2 = jnp.kron(att_W2.T, eye8)
    b1f = jnp.repeat(att_b1, H)[None, :]
    b2f = jnp.repeat(att_b2, H)[None, :]
    WqT, WkT, WvT = Wq.T, Wk.T, Wv.T
    WaT = eu_W1[:, :D].T
    WbT = eu_W1[:, D:2 * D].T
    WcT = eu_W1[:, 2 * D:3 * D].T
    WdT = eu_W1[:, 3 * D:].T
    W1pT = dm_W1[:, :3].T
    w1dr = dm_W1[:, 3][None, :]
    n1aT = nu_W1[:, :D].T
    n1bT = nu_W1[:, D:].T
    eaTa = ea_W[:, :D].T
    eaTb = ea_W[:, D:].T

    row2 = lambda a: a[None, :].astype(f32)
    bq2, bk2, bv2 = row2(bq), row2(bk), row2(bv)
    dm_b1r, dm_b2r = row2(dm_b1), row2(dm_b2)
    eu_b1r, eu_b2r = row2(eu_b1), row2(eu_b2)
    nu_b1r, nu_b2r = row2(nu_b1), row2(nu_b2)
    ea_br = row2(ea_b)

    eblk = lambda w: pl.BlockSpec((BLK, w), lambda i: (i, 0))
    edge_in_specs = [eblk(D), eblk(D), eblk(D), eblk(D), eblk(3), eblk(3)]
    w_specs_a = [
        _full(WqT.shape), _full(bq2.shape), _full(WkT.shape), _full(bk2.shape),
        _full(M1q.shape), _full(M1k.shape), _full(b1f.shape),
        _full(WaT.shape), _full(WbT.shape), _full(WcT.shape), _full(WdT.shape),
        _full(eu_b1r.shape),
        _full(W1pT.shape), _full(w1dr.shape), _full(dm_b1r.shape),
    ]
    edge_args = (x_i, x_j, edge_feature, rev_ef, p_i, p_j)
    w_args_a = (WqT, bq2, WkT, bk2, M1q, M1k, b1f,
                WaT, WbT, WcT, WdT, eu_b1r,
                W1pT, w1dr, dm_b1r)

    s_hdm, s_a, s_he = pl.pallas_call(
        _pass_a,
        grid=(GRID,),
        in_specs=edge_in_specs + w_specs_a,
        out_specs=[_full((8, 32)), _full((8, 256)), _full((8, 384))],
        out_shape=[jax.ShapeDtypeStruct((8, 32), f32),
                   jax.ShapeDtypeStruct((8, 256), f32),
                   jax.ShapeDtypeStruct((8, 384), f32)],
    )(*edge_args, *w_args_a)

    w_specs_b = w_specs_a + [
        _full((8, 32)), _full((8, 256)), _full((8, 384)),
        _full(M2.shape), _full(b2f.shape), _full(WvT.shape), _full(bv2.shape),
        _full((32, 1)), _full(dm_b2r.shape), _full((3 * D, D)),
        _full(eu_b2r.shape),
    ]
    msg, ue, prob_flat = pl.pallas_call(
        _pass_b,
        grid=(GRID,),
        in_specs=edge_in_specs + w_specs_b,
        out_specs=[eblk(D), eblk(D), eblk(D)],
        out_shape=[jax.ShapeDtypeStruct((E, D), f32),
                   jax.ShapeDtypeStruct((E, D), f32),
                   jax.ShapeDtypeStruct((E, D), f32)],
    )(*edge_args, *w_args_a, s_hdm, s_a, s_he,
      M2, b2f, WvT, bv2, dm_W2.T, dm_b2r, eu_W2.T, eu_b2r)

    agg = jax.ops.segment_max(msg, row, num_segments=N)
    agg = jnp.where(jnp.isfinite(agg), agg, 0.0)
    ones = jnp.ones((E,), f32)
    out_sum = jax.ops.segment_sum(ue, row, num_segments=N)
    out_cnt = jax.ops.segment_sum(ones, row, num_segments=N)[:, None]
    in_sum = jax.ops.segment_sum(ue, col, num_segments=N)
    in_cnt = jax.ops.segment_sum(ones, col, num_segments=N)[:, None]

    final_node = pl.pallas_call(
        _node_kernel,
        out_shape=jax.ShapeDtypeStruct((N, D), f32),
    )(x, agg, out_sum, out_cnt, in_sum, in_cnt,
      n1aT, n1bT, nu_b1r, nu_W2.T, nu_b2r, eaTa, eaTb, ea_br)

    prob = prob_flat.reshape(E, DP, H)
    return final_node, ue, prob


# speedup vs baseline: 1.1014x; 1.1014x over previous
import functools

import jax
import jax.numpy as jnp
from jax import lax
from jax.experimental import pallas as pl

N = 10000
E = 160000
D = 128
H = 8
DP = 16
TEMP = 4.0
BLK = 4000
GRID = E // BLK


def _group_mat(width, stride_same):
    ci = lax.broadcasted_iota(jnp.int32, (width, width), 0)
    cj = lax.broadcasted_iota(jnp.int32, (width, width), 1)
    if stride_same == "mod8":
        gi, gj = ci - (ci // 8) * 8, cj - (cj // 8) * 8
    else:
        gi, gj = ci // 8, cj // 8
    return (gi == gj).astype(jnp.float32)


def _edge_preacts(xpi, xpj, ef, rev,
                  WqT, bq, WkT, bk, M1q, M1k, b1f,
                  WaT, WbT, WcT, WdT, eu_b1,
                  W1pT, w1dr, dm_b1):
    dot = functools.partial(jnp.dot, preferred_element_type=jnp.float32)
    xi, pi = xpi[:, :D], xpi[:, D:]
    xj, pj = xpj[:, :D], xpj[:, D:]
    qlin = dot(xi, WqT) + bq
    klin = dot(ef, WkT) + bk
    a_pre = dot(qlin, M1q) + dot(klin, M1k) + b1f
    he_pre = (dot(xi, WaT) + dot(ef, WbT) + dot(rev, WcT) + dot(xj, WdT)
              + eu_b1)
    diff = pi - pj
    dist = jnp.sqrt(jnp.sum(diff * diff, axis=-1, keepdims=True) + 1e-12)
    hdm_pre = dot(diff, W1pT) + dist * w1dr + dm_b1
    return qlin, klin, a_pre, he_pre, hdm_pre


def _pass_a(xi_ref, xj_ref, ef_ref, rev_ref,
            WqT, bq, WkT, bk, M1q, M1k, b1f,
            WaT, WbT, WcT, WdT, eu_b1,
            W1pT, w1dr, dm_b1,
            s_hdm_ref, s_a_ref, s_he_ref):
    i = pl.program_id(0)

    @pl.when(i == 0)
    def _():
        s_hdm_ref[...] = jnp.zeros_like(s_hdm_ref)
        s_a_ref[...] = jnp.zeros_like(s_a_ref)
        s_he_ref[...] = jnp.zeros_like(s_he_ref)

    _, _, a_pre, he_pre, hdm_pre = _edge_preacts(
        xi_ref[...], xj_ref[...], ef_ref[...], rev_ref[...],
        WqT[...], bq[...], WkT[...], bk[...], M1q[...], M1k[...], b1f[...],
        WaT[...], WbT[...], WcT[...], WdT[...], eu_b1[...],
        W1pT[...], w1dr[...], dm_b1[...])

    s_hdm_ref[0:1, :] += jnp.sum(hdm_pre, axis=0, keepdims=True)
    s_hdm_ref[1:2, :] += jnp.sum(hdm_pre * hdm_pre, axis=0, keepdims=True)
    s_a_ref[0:1, :] += jnp.sum(a_pre, axis=0, keepdims=True)
    s_a_ref[1:2, :] += jnp.sum(a_pre * a_pre, axis=0, keepdims=True)
    s_he_ref[0:1, :] += jnp.sum(he_pre, axis=0, keepdims=True)
    s_he_ref[1:2, :] += jnp.sum(he_pre * he_pre, axis=0, keepdims=True)


def _pass_b(xi_ref, xj_ref, ef_ref, rev_ref,
            WqT, bq, WkT, bk, M1q, M1k, b1f,
            WaT, WbT, WcT, WdT, eu_b1,
            W1pT, w1dr, dm_b1,
            s_hdm, s_a, s_he,
            M2, b2f, WvT, bv, dm_W2T, dm_b2, eu_W2T, eu_b2,
            msg_ref, ue_ref, prob_ref):
    dot = functools.partial(jnp.dot, preferred_element_type=jnp.float32)
    qlin, klin, a_pre, he_pre, hdm_pre = _edge_preacts(
        xi_ref[...], xj_ref[...], ef_ref[...], rev_ref[...],
        WqT[...], bq[...], WkT[...], bk[...], M1q[...], M1k[...], b1f[...],
        WaT[...], WbT[...], WcT[...], WdT[...], eu_b1[...],
        W1pT[...], w1dr[...], dm_b1[...])

    inv_e = 1.0 / E
    hdm_m = s_hdm[0:1, :] * inv_e
    hdm_v = s_hdm[1:2, :] * inv_e - hdm_m * hdm_m
    hdm = jax.nn.relu((hdm_pre - hdm_m) * lax.rsqrt(hdm_v + 1e-5))
    dmask = jax.nn.sigmoid(dot(hdm, dm_W2T[...]) + dm_b2[...])

    g8 = _group_mat(256, "div8")
    a_cnt = 1.0 / (E * H)
    a_m = dot(s_a[0:1, :], g8) * a_cnt
    a_v = dot(s_a[1:2, :], g8) * a_cnt - a_m * a_m
    a_bn = jax.nn.relu((a_pre - a_m) * lax.rsqrt(a_v + 1e-5))
    att = dot(a_bn, M2[...]) + b2f[...]

    gh = _group_mat(128, "mod8")
    gmean = dot(att, gh) * (1.0 / DP)
    u = jnp.exp((att - gmean) * (1.0 / TEMP))
    denom = dot(u, gh)
    prob = u / denom
    prob_ref[...] = prob

    vlin = dot(xj_ref[:, :D], WvT[...]) + bv[...]
    msg_ref[...] = prob * vlin * dmask

    he_m = s_he[0:1, :] * inv_e
    he_v = s_he[1:2, :] * inv_e - he_m * he_m
    he = jax.nn.relu((he_pre - he_m) * lax.rsqrt(he_v + 1e-5))
    ue_ref[...] = dot(he, eu_W2T[...]) + eu_b2[...]


def _node_kernel(x_ref, agg_ref, osum_ref, ocnt_ref, isum_ref, icnt_ref,
                 n1aT, n1bT, nu_b1, nu_W2T, nu_b2,
                 eaTa, eaTb, ea_b,
                 out_ref):
    dot = functools.partial(jnp.dot, preferred_element_type=jnp.float32)
    hn_pre = dot(x_ref[...], n1aT[...]) + dot(agg_ref[...], n1bT[...]) \
        + nu_b1[...]
    m = jnp.mean(hn_pre, axis=0, keepdims=True)
    v = jnp.mean(hn_pre * hn_pre, axis=0, keepdims=True) - m * m
    hn = jax.nn.relu((hn_pre - m) * lax.rsqrt(v + 1e-5))
    un = dot(hn, nu_W2T[...]) + nu_b2[...]
    om = osum_ref[...] / jnp.maximum(ocnt_ref[...], 1.0)
    im = isum_ref[...] / jnp.maximum(icnt_ref[...], 1.0)
    gate = jax.nn.sigmoid(dot(om, eaTa[...]) + dot(im, eaTb[...]) + ea_b[...])
    out_ref[...] = jax.nn.relu(un) * gate


def _full(shape):
    return pl.BlockSpec(shape, lambda i: (0,) * len(shape))


def kernel(x, edge_feature, node_positions, Wq, bq, Wk, bk, Wv, bv,
           dm_W1, dm_b1, dm_W2, dm_b2, att_W1, att_b1, att_W2, att_b2,
           eu_W1, eu_b1, eu_W2, eu_b2, ea_W, ea_b,
           nu_W1, nu_b1, nu_W2, nu_b2, edge_index):
    f32 = jnp.float32
    row = edge_index[0]
    col = edge_index[1]
    keys = row.astype(jnp.int64) * N + col.astype(jnp.int64)
    order = jnp.argsort(keys)
    skeys = keys[order]
    rkeys = col.astype(jnp.int64) * N + row.astype(jnp.int64)
    pos = jnp.clip(jnp.searchsorted(skeys, rkeys), 0, E - 1)
    found = skeys[pos] == rkeys
    rev_ef = jnp.where(found[:, None], edge_feature[jnp.clip(order[pos], 0, E - 1)], 0.0)
    xp = jnp.concatenate([x, node_positions], axis=1)
    xp_i = xp[row]
    xp_j = xp[col]

    eye8 = jnp.eye(8, dtype=f32)
    M1q = jnp.kron(att_W1[:, :DP].T, eye8)
    M1k = jnp.kron(att_W1[:, DP:].T, eye8)
    M2 = jnp.kron(att_W2.T, eye8)
    b1f = jnp.repeat(att_b1, H)[None, :]
    b2f = jnp.repeat(att_b2, H)[None, :]
    WqT, WkT, WvT = Wq.T, Wk.T, Wv.T
    WaT = eu_W1[:, :D].T
    WbT = eu_W1[:, D:2 * D].T
    WcT = eu_W1[:, 2 * D:3 * D].T
    WdT = eu_W1[:, 3 * D:].T
    W1pT = dm_W1[:, :3].T
    w1dr = dm_W1[:, 3][None, :]
    n1aT = nu_W1[:, :D].T
    n1bT = nu_W1[:, D:].T
    eaTa = ea_W[:, :D].T
    eaTb = ea_W[:, D:].T

    row2 = lambda a: a[None, :].astype(f32)
    bq2, bk2, bv2 = row2(bq), row2(bk), row2(bv)
    dm_b1r, dm_b2r = row2(dm_b1), row2(dm_b2)
    eu_b1r, eu_b2r = row2(eu_b1), row2(eu_b2)
    nu_b1r, nu_b2r = row2(nu_b1), row2(nu_b2)
    ea_br = row2(ea_b)

    eblk = lambda w: pl.BlockSpec((BLK, w), lambda i: (i, 0))
    edge_in_specs = [eblk(D + 3), eblk(D + 3), eblk(D), eblk(D)]
    w_specs_a = [
        _full(WqT.shape), _full(bq2.shape), _full(WkT.shape), _full(bk2.shape),
        _full(M1q.shape), _full(M1k.shape), _full(b1f.shape),
        _full(WaT.shape), _full(WbT.shape), _full(WcT.shape), _full(WdT.shape),
        _full(eu_b1r.shape),
        _full(W1pT.shape), _full(w1dr.shape), _full(dm_b1r.shape),
    ]
    edge_args = (xp_i, xp_j, edge_feature, rev_ef)
    w_args_a = (WqT, bq2, WkT, bk2, M1q, M1k, b1f,
                WaT, WbT, WcT, WdT, eu_b1r,
                W1pT, w1dr, dm_b1r)

    s_hdm, s_a, s_he = pl.pallas_call(
        _pass_a,
        grid=(GRID,),
        in_specs=edge_in_specs + w_specs_a,
        out_specs=[_full((8, 32)), _full((8, 256)), _full((8, 384))],
        out_shape=[jax.ShapeDtypeStruct((8, 32), f32),
                   jax.ShapeDtypeStruct((8, 256), f32),
                   jax.ShapeDtypeStruct((8, 384), f32)],
    )(*edge_args, *w_args_a)

    w_specs_b = w_specs_a + [
        _full((8, 32)), _full((8, 256)), _full((8, 384)),
        _full(M2.shape), _full(b2f.shape), _full(WvT.shape), _full(bv2.shape),
        _full((32, 1)), _full(dm_b2r.shape), _full((3 * D, D)),
        _full(eu_b2r.shape),
    ]
    msg, ue, prob_flat = pl.pallas_call(
        _pass_b,
        grid=(GRID,),
        in_specs=edge_in_specs + w_specs_b,
        out_specs=[eblk(D), eblk(D), eblk(D)],
        out_shape=[jax.ShapeDtypeStruct((E, D), f32),
                   jax.ShapeDtypeStruct((E, D), f32),
                   jax.ShapeDtypeStruct((E, D), f32)],
    )(*edge_args, *w_args_a, s_hdm, s_a, s_he,
      M2, b2f, WvT, bv2, dm_W2.T, dm_b2r, eu_W2.T, eu_b2r)

    agg = jax.ops.segment_max(msg, row, num_segments=N)
    agg = jnp.where(jnp.isfinite(agg), agg, 0.0)
    ones = jnp.ones((E,), f32)
    out_sum = jax.ops.segment_sum(ue, row, num_segments=N)
    out_cnt = jax.ops.segment_sum(ones, row, num_segments=N)[:, None]
    in_sum = jax.ops.segment_sum(ue, col, num_segments=N)
    in_cnt = jax.ops.segment_sum(ones, col, num_segments=N)[:, None]

    final_node = pl.pallas_call(
        _node_kernel,
        out_shape=jax.ShapeDtypeStruct((N, D), f32),
    )(x, agg, out_sum, out_cnt, in_sum, in_cnt,
      n1aT, n1bT, nu_b1r, nu_W2.T, nu_b2r, eaTa, eaTb, ea_br)

    prob = prob_flat.reshape(E, DP, H)
    return final_node, ue, prob


# speedup vs baseline: 1.1027x; 1.0011x over previous
import functools

import jax
import jax.numpy as jnp
from jax import lax
from jax.experimental import pallas as pl

N = 10000
E = 160000
D = 128
H = 8
DP = 16
TEMP = 4.0
BLK = 4000
GRID = E // BLK


def _group_mat(width, stride_same):
    ci = lax.broadcasted_iota(jnp.int32, (width, width), 0)
    cj = lax.broadcasted_iota(jnp.int32, (width, width), 1)
    if stride_same == "mod8":
        gi, gj = ci - (ci // 8) * 8, cj - (cj // 8) * 8
    else:
        gi, gj = ci // 8, cj // 8
    return (gi == gj).astype(jnp.float32)


def _edge_preacts(xpi, xpj, ef, rev,
                  WqT, bq, WkT, bk, M1q, M1k, b1f,
                  WaT, WbT, WcT, WdT, eu_b1,
                  W1pT, w1dr, dm_b1):
    dot = functools.partial(jnp.dot, preferred_element_type=jnp.float32)
    xi, pi = xpi[:, :D], xpi[:, D:]
    xj, pj = xpj[:, :D], xpj[:, D:]
    qlin = dot(xi, WqT) + bq
    klin = dot(ef, WkT) + bk
    a_pre = dot(qlin, M1q) + dot(klin, M1k) + b1f
    he_pre = (dot(xi, WaT) + dot(ef, WbT) + dot(rev, WcT) + dot(xj, WdT)
              + eu_b1)
    diff = pi - pj
    dist = jnp.sqrt(jnp.sum(diff * diff, axis=-1, keepdims=True) + 1e-12)
    hdm_pre = dot(diff, W1pT) + dist * w1dr + dm_b1
    return qlin, klin, a_pre, he_pre, hdm_pre


def _pass_a(xi_ref, xj_ref, ef_ref, rev_ref,
            WqT, bq, WkT, bk, M1q, M1k, b1f,
            WaT, WbT, WcT, WdT, eu_b1,
            W1pT, w1dr, dm_b1,
            s_hdm_ref, s_a_ref, s_he_ref):
    i = pl.program_id(0)

    @pl.when(i == 0)
    def _():
        s_hdm_ref[...] = jnp.zeros_like(s_hdm_ref)
        s_a_ref[...] = jnp.zeros_like(s_a_ref)
        s_he_ref[...] = jnp.zeros_like(s_he_ref)

    _, _, a_pre, he_pre, hdm_pre = _edge_preacts(
        xi_ref[...], xj_ref[...], ef_ref[...], rev_ref[...],
        WqT[...], bq[...], WkT[...], bk[...], M1q[...], M1k[...], b1f[...],
        WaT[...], WbT[...], WcT[...], WdT[...], eu_b1[...],
        W1pT[...], w1dr[...], dm_b1[...])

    s_hdm_ref[0:1, :] += jnp.sum(hdm_pre, axis=0, keepdims=True)
    s_hdm_ref[1:2, :] += jnp.sum(hdm_pre * hdm_pre, axis=0, keepdims=True)
    s_a_ref[0:1, :] += jnp.sum(a_pre, axis=0, keepdims=True)
    s_a_ref[1:2, :] += jnp.sum(a_pre * a_pre, axis=0, keepdims=True)
    s_he_ref[0:1, :] += jnp.sum(he_pre, axis=0, keepdims=True)
    s_he_ref[1:2, :] += jnp.sum(he_pre * he_pre, axis=0, keepdims=True)


def _pass_b(xi_ref, xj_ref, ef_ref, rev_ref,
            WqT, bq, WkT, bk, M1q, M1k, b1f,
            WaT, WbT, WcT, WdT, eu_b1,
            W1pT, w1dr, dm_b1,
            s_hdm, s_a, s_he,
            M2, b2f, WvT, bv, dm_W2T, dm_b2, eu_W2T, eu_b2,
            msg_ref, ue_ref, prob_ref):
    dot = functools.partial(jnp.dot, preferred_element_type=jnp.float32)
    qlin, klin, a_pre, he_pre, hdm_pre = _edge_preacts(
        xi_ref[...], xj_ref[...], ef_ref[...], rev_ref[...],
        WqT[...], bq[...], WkT[...], bk[...], M1q[...], M1k[...], b1f[...],
        WaT[...], WbT[...], WcT[...], WdT[...], eu_b1[...],
        W1pT[...], w1dr[...], dm_b1[...])

    inv_e = 1.0 / E
    hdm_m = s_hdm[0:1, :] * inv_e
    hdm_v = s_hdm[1:2, :] * inv_e - hdm_m * hdm_m
    hdm = jax.nn.relu((hdm_pre - hdm_m) * lax.rsqrt(hdm_v + 1e-5))
    dmask = jax.nn.sigmoid(dot(hdm, dm_W2T[...]) + dm_b2[...])

    g8 = _group_mat(256, "div8")
    a_cnt = 1.0 / (E * H)
    a_m = dot(s_a[0:1, :], g8) * a_cnt
    a_v = dot(s_a[1:2, :], g8) * a_cnt - a_m * a_m
    a_bn = jax.nn.relu((a_pre - a_m) * lax.rsqrt(a_v + 1e-5))
    att = dot(a_bn, M2[...]) + b2f[...]

    gh = _group_mat(128, "mod8")
    gmean = dot(att, gh) * (1.0 / DP)
    u = jnp.exp((att - gmean) * (1.0 / TEMP))
    denom = dot(u, gh)
    prob = u / denom
    prob_ref[...] = prob

    vlin = dot(xj_ref[:, :D], WvT[...]) + bv[...]
    msg_ref[...] = prob * vlin * dmask

    he_m = s_he[0:1, :] * inv_e
    he_v = s_he[1:2, :] * inv_e - he_m * he_m
    he = jax.nn.relu((he_pre - he_m) * lax.rsqrt(he_v + 1e-5))
    ue_ref[...] = dot(he, eu_W2T[...]) + eu_b2[...]


def _node_kernel(x_ref, agg_ref, osum_ref, ocnt_ref, isum_ref, icnt_ref,
                 n1aT, n1bT, nu_b1, nu_W2T, nu_b2,
                 eaTa, eaTb, ea_b,
                 out_ref):
    dot = functools.partial(jnp.dot, preferred_element_type=jnp.float32)
    hn_pre = dot(x_ref[...], n1aT[...]) + dot(agg_ref[...], n1bT[...]) \
        + nu_b1[...]
    m = jnp.mean(hn_pre, axis=0, keepdims=True)
    v = jnp.mean(hn_pre * hn_pre, axis=0, keepdims=True) - m * m
    hn = jax.nn.relu((hn_pre - m) * lax.rsqrt(v + 1e-5))
    un = dot(hn, nu_W2T[...]) + nu_b2[...]
    om = osum_ref[...] / jnp.maximum(ocnt_ref[...], 1.0)
    im = isum_ref[...] / jnp.maximum(icnt_ref[...], 1.0)
    gate = jax.nn.sigmoid(dot(om, eaTa[...]) + dot(im, eaTb[...]) + ea_b[...])
    out_ref[...] = jax.nn.relu(un) * gate


def _full(shape):
    return pl.BlockSpec(shape, lambda i: (0,) * len(shape))


def kernel(x, edge_feature, node_positions, Wq, bq, Wk, bk, Wv, bv,
           dm_W1, dm_b1, dm_W2, dm_b2, att_W1, att_b1, att_W2, att_b2,
           eu_W1, eu_b1, eu_W2, eu_b2, ea_W, ea_b,
           nu_W1, nu_b1, nu_W2, nu_b2, edge_index):
    f32 = jnp.float32
    row = edge_index[0]
    col = edge_index[1]
    keys = row.astype(jnp.int64) * N + col.astype(jnp.int64)
    order = jnp.argsort(keys)
    skeys = keys[order]
    rkeys = col.astype(jnp.int64) * N + row.astype(jnp.int64)
    pos = jnp.clip(jnp.searchsorted(skeys, rkeys), 0, E - 1)
    found = skeys[pos] == rkeys
    rev_ef = jnp.where(found[:, None], edge_feature[jnp.clip(order[pos], 0, E - 1)], 0.0)
    xp = jnp.concatenate([x, node_positions], axis=1)
    xp_i = xp[row]
    xp_j = xp[col]

    eye8 = jnp.eye(8, dtype=f32)
    M1q = jnp.kron(att_W1[:, :DP].T, eye8)
    M1k = jnp.kron(att_W1[:, DP:].T, eye8)
    M2 = jnp.kron(att_W2.T, eye8)
    b1f = jnp.repeat(att_b1, H)[None, :]
    b2f = jnp.repeat(att_b2, H)[None, :]
    WqT, WkT, WvT = Wq.T, Wk.T, Wv.T
    WaT = eu_W1[:, :D].T
    WbT = eu_W1[:, D:2 * D].T
    WcT = eu_W1[:, 2 * D:3 * D].T
    WdT = eu_W1[:, 3 * D:].T
    W1pT = dm_W1[:, :3].T
    w1dr = dm_W1[:, 3][None, :]
    n1aT = nu_W1[:, :D].T
    n1bT = nu_W1[:, D:].T
    eaTa = ea_W[:, :D].T
    eaTb = ea_W[:, D:].T

    row2 = lambda a: a[None, :].astype(f32)
    bq2, bk2, bv2 = row2(bq), row2(bk), row2(bv)
    dm_b1r, dm_b2r = row2(dm_b1), row2(dm_b2)
    eu_b1r, eu_b2r = row2(eu_b1), row2(eu_b2)
    nu_b1r, nu_b2r = row2(nu_b1), row2(nu_b2)
    ea_br = row2(ea_b)

    eblk = lambda w: pl.BlockSpec((BLK, w), lambda i: (i, 0))
    edge_in_specs = [eblk(D + 3), eblk(D + 3), eblk(D), eblk(D)]
    w_specs_a = [
        _full(WqT.shape), _full(bq2.shape), _full(WkT.shape), _full(bk2.shape),
        _full(M1q.shape), _full(M1k.shape), _full(b1f.shape),
        _full(WaT.shape), _full(WbT.shape), _full(WcT.shape), _full(WdT.shape),
        _full(eu_b1r.shape),
        _full(W1pT.shape), _full(w1dr.shape), _full(dm_b1r.shape),
    ]
    edge_args = (xp_i, xp_j, edge_feature, rev_ef)
    w_args_a = (WqT, bq2, WkT, bk2, M1q, M1k, b1f,
                WaT, WbT, WcT, WdT, eu_b1r,
                W1pT, w1dr, dm_b1r)

    s_hdm, s_a, s_he = pl.pallas_call(
        _pass_a,
        grid=(GRID,),
        in_specs=edge_in_specs + w_specs_a,
        out_specs=[_full((8, 32)), _full((8, 256)), _full((8, 384))],
        out_shape=[jax.ShapeDtypeStruct((8, 32), f32),
                   jax.ShapeDtypeStruct((8, 256), f32),
                   jax.ShapeDtypeStruct((8, 384), f32)],
    )(*edge_args, *w_args_a)

    w_specs_b = w_specs_a + [
        _full((8, 32)), _full((8, 256)), _full((8, 384)),
        _full(M2.shape), _full(b2f.shape), _full(WvT.shape), _full(bv2.shape),
        _full((32, 1)), _full(dm_b2r.shape), _full((3 * D, D)),
        _full(eu_b2r.shape),
    ]
    msg, ue, prob_flat = pl.pallas_call(
        _pass_b,
        grid=(GRID,),
        in_specs=edge_in_specs + w_specs_b,
        out_specs=[eblk(D), eblk(D), eblk(D)],
        out_shape=[jax.ShapeDtypeStruct((E, D), f32),
                   jax.ShapeDtypeStruct((E, D), f32),
                   jax.ShapeDtypeStruct((E, D), f32)],
    )(*edge_args, *w_args_a, s_hdm, s_a, s_he,
      M2, b2f, WvT, bv2, dm_W2.T, dm_b2r, eu_W2.T, eu_b2r)

    agg = jax.ops.segment_max(msg, row, num_segments=N)
    agg = jnp.where(jnp.isfinite(agg), agg, 0.0)
    ue1 = jnp.concatenate([ue, jnp.ones((E, 1), f32)], axis=1)
    osum1 = jax.ops.segment_sum(ue1, row, num_segments=N)
    isum1 = jax.ops.segment_sum(ue1, col, num_segments=N)
    out_sum, out_cnt = osum1[:, :D], osum1[:, D:]
    in_sum, in_cnt = isum1[:, :D], isum1[:, D:]

    final_node = pl.pallas_call(
        _node_kernel,
        out_shape=jax.ShapeDtypeStruct((N, D), f32),
    )(x, agg, out_sum, out_cnt, in_sum, in_cnt,
      n1aT, n1bT, nu_b1r, nu_W2.T, nu_b2r, eaTa, eaTb, ea_br)

    prob = prob_flat.reshape(E, DP, H)
    return final_node, ue, prob
